# baseline (speedup 1.0000x reference)
import functools

import jax
import jax.numpy as jnp
from jax.experimental import pallas as pl
from jax.experimental.pallas import tpu as pltpu

_NUM_CLASSES = 2
_THRESHOLD = 0.35
_NEGPOS_RATIO = 7
_VAR0, _VAR1 = 0.1, 0.2


def _smooth_l1(d):
    a = jnp.abs(d)
    return jnp.where(a < 1.0, 0.5 * d * d, a - 0.5)


def _match_kernel(locT_ref, landmT_ref, tgt_ref, tgtT_ref,
                  priorsT_ref, pos_ref, np_ref, ll_ref, llm_ref,
                  *, num_priors: int, num_obj: int, pack: int):
    P = num_priors
    O = num_obj
    R = pack * O

    pr = priorsT_ref[0]
    pcx, pcy, pw, ph = pr[0:1, :], pr[1:2, :], pr[2:3, :], pr[3:4, :]
    px1 = pcx - pw * 0.5
    py1 = pcy - ph * 0.5
    px2 = pcx + pw * 0.5
    py2 = pcy + ph * 0.5
    area_b = (px2 - px1) * (py2 - py1)

    tg = tgt_ref[:].reshape(R, 15)
    tx1 = tg[:, 0:1]
    ty1 = tg[:, 1:2]
    tx2 = tg[:, 2:3]
    ty2 = tg[:, 3:4]
    area_a = (tx2 - tx1) * (ty2 - ty1)

    ix1 = jnp.maximum(tx1, px1)
    iy1 = jnp.maximum(ty1, py1)
    ix2 = jnp.minimum(tx2, px2)
    iy2 = jnp.minimum(ty2, py2)
    inter = jnp.maximum(ix2 - ix1, 0.0) * jnp.maximum(iy2 - iy1, 0.0)
    ov = inter / (area_a + area_b - inter)

    idx_p = jax.lax.broadcasted_iota(jnp.int32, (R, P), 1)
    idx_t = jax.lax.broadcasted_iota(jnp.int32, (O, P), 0)

    bpo = jnp.max(ov, axis=1, keepdims=True)
    bp_idx = jnp.min(jnp.where(ov == bpo, idx_p, P), axis=1, keepdims=True)
    valid = bpo >= 0.2
    is_bp = bp_idx == idx_p

    row10 = jax.lax.broadcasted_iota(jnp.int32, (10, P), 0)
    is_x = (row10 % 2) == 0
    cxy10 = jnp.where(is_x, pcx, pcy)
    wh10 = jnp.where(is_x, pw, ph)
    pcxy = pr[0:2, :]
    pwh = pr[2:4, :]

    for i in range(pack):
        ovi = ov[O * i:O * (i + 1), :]
        validi = valid[O * i:O * (i + 1), :]
        is_bpi = is_bp[O * i:O * (i + 1), :]
        any_valid = jnp.max(validi.astype(jnp.float32)) > 0.0

        bto = jnp.max(ovi, axis=0, keepdims=True)
        bt_idx = jnp.min(jnp.where(ovi == bto, idx_t, O), axis=0,
                         keepdims=True)

        has_valid_ov = jnp.max(jnp.where(is_bpi & validi, 1.0, 0.0), axis=0,
                               keepdims=True) > 0.0
        new_ov = jnp.where(has_valid_ov, 2.0, bto)
        t_over = jnp.max(jnp.where(is_bpi, idx_t, -1), axis=0, keepdims=True)
        new_idx = jnp.where(t_over >= 0, t_over, bt_idx)

        onehot = (new_idx == idx_t).astype(jnp.float32)
        matched = jnp.dot(tgtT_ref[i], onehot,
                          preferred_element_type=jnp.float32)

        label = matched[14:15, :]
        conf = jnp.where(new_ov < _THRESHOLD, 0.0, label)
        conf = jnp.where(any_valid, conf, 0.0)
        pos = conf > 0.5
        posf = pos.astype(jnp.float32)

        m01 = matched[0:2, :]
        m23 = matched[2:4, :]
        g_cxy = ((m01 + m23) * 0.5 - pcxy) / (_VAR0 * pwh)
        g_wh = jnp.log((m23 - m01) / pwh) / _VAR1
        locT = locT_ref[i]
        loss_l = jnp.sum(posf * (_smooth_l1(locT[0:2, :] - g_cxy)
                                 + _smooth_l1(locT[2:4, :] - g_wh)))

        g_lm = (matched[4:14, :] - cxy10) / (_VAR0 * wh10)
        loss_lm = jnp.sum(posf * _smooth_l1(landmT_ref[i] - g_lm))

        pos_ref[i] = posf
        np_ref[i] = jnp.full((1, 1), jnp.sum(posf), jnp.float32)
        ll_ref[i] = jnp.full((1, 1), loss_l, jnp.float32)
        llm_ref[i] = jnp.full((1, 1), loss_lm, jnp.float32)


def _mine_kernel(pos_ref, c0_ref, c1_ref, np_ref, ll_ref, llm_ref,
                 out_l_ref, out_c_ref, out_lm_ref,
                 *, batch: int, num_priors: int):
    P = num_priors
    posf = pos_ref[:]
    pos = posf > 0.5
    c0 = c0_ref[:]
    c1 = c1_ref[:]
    m = jnp.maximum(c0, c1)
    lse_tail = jnp.log(1.0 + jnp.exp(-jnp.abs(c0 - c1)))
    picked = jnp.where(pos, c1, c0)
    ce = jnp.maximum((m - picked) + lse_tail, 0.0)
    cep = jnp.sum(posf * ce)
    x = jnp.where(pos, 0.0, ce)
    xi = jax.lax.bitcast_convert_type(x, jnp.int32)
    npos = np_ref[:, :, 0]
    k = jnp.minimum(npos.astype(jnp.int32) * _NEGPOS_RATIO, P - 1)

    def body(i, t):
        cand = t | (jnp.int32(1) << (jnp.int32(30) - i))
        cnt = jnp.sum((xi >= cand).astype(jnp.int32), axis=1, keepdims=True)
        return jnp.where(cnt >= k, cand, t)

    t = jax.lax.fori_loop(0, 31, body, jnp.zeros((batch, 1), jnp.int32))
    v = jax.lax.bitcast_convert_type(t, jnp.float32)
    gt = x > v
    sum_gt = jnp.sum(jnp.where(gt, x, 0.0), axis=1, keepdims=True)
    cnt_gt = jnp.sum(gt.astype(jnp.int32), axis=1, keepdims=True)
    row = sum_gt + (k - cnt_gt).astype(jnp.float32) * v
    mining = jnp.sum(jnp.where(k > 0, row, 0.0))

    n = jnp.maximum(jnp.sum(npos), 1.0)
    out_l_ref[:, :] = jnp.full((1, 1), jnp.sum(ll_ref[:]) / n, jnp.float32)
    out_c_ref[:, :] = jnp.full((1, 1), (cep + mining) / n, jnp.float32)
    out_lm_ref[:, :] = jnp.full((1, 1), jnp.sum(llm_ref[:]) / n, jnp.float32)


@jax.jit
def kernel(locations_data, confidence_data, landmark_data, targets, priors):
    B, P, _ = locations_data.shape
    O = targets.shape[1]

    locT = locations_data.transpose(0, 2, 1)
    landmT = landmark_data.transpose(0, 2, 1)
    tgtT = targets.transpose(0, 2, 1)
    priorsT = priors.T[None]
    c0 = confidence_data[:, :, 0]
    c1 = confidence_data[:, :, 1]

    PACK = 8
    s111 = jax.ShapeDtypeStruct((B, 1, 1), jnp.float32)
    k1 = functools.partial(_match_kernel, num_priors=P, num_obj=O, pack=PACK)
    posf, npos, ll, llm = pl.pallas_call(
        k1,
        grid=(B // PACK,),
        in_specs=[
            pl.BlockSpec((PACK, 4, P), lambda b: (b, 0, 0)),
            pl.BlockSpec((PACK, 10, P), lambda b: (b, 0, 0)),
            pl.BlockSpec((PACK, O, 15), lambda b: (b, 0, 0)),
            pl.BlockSpec((PACK, 15, O), lambda b: (b, 0, 0)),
            pl.BlockSpec((1, 4, P), lambda b: (0, 0, 0)),
        ],
        out_specs=[
            pl.BlockSpec((PACK, 1, P), lambda b: (b, 0, 0)),
            pl.BlockSpec((PACK, 1, 1), lambda b: (b, 0, 0)),
            pl.BlockSpec((PACK, 1, 1), lambda b: (b, 0, 0)),
            pl.BlockSpec((PACK, 1, 1), lambda b: (b, 0, 0)),
        ],
        out_shape=[
            jax.ShapeDtypeStruct((B, 1, P), jnp.float32),
            s111, s111, s111,
        ],
        compiler_params=pltpu.CompilerParams(
            dimension_semantics=("parallel",)),
    )(locT, landmT, targets, tgtT, priorsT)

    scalar = jax.ShapeDtypeStruct((1, 1), jnp.float32)
    k2 = functools.partial(_mine_kernel, batch=B, num_priors=P)
    out_l, out_c, out_lm = pl.pallas_call(
        k2,
        out_shape=[scalar, scalar, scalar],
    )(posf.reshape(B, P), c0, c1, npos, ll, llm)
    return out_l[0, 0], out_c[0, 0], out_lm[0, 0]


# speedup vs baseline: 1.0696x; 1.0696x over previous
import functools

import jax
import jax.numpy as jnp
from jax.experimental import pallas as pl
from jax.experimental.pallas import tpu as pltpu

_NUM_CLASSES = 2
_THRESHOLD = 0.35
_NEGPOS_RATIO = 7
_VAR0, _VAR1 = 0.1, 0.2


def _smooth_l1(d):
    a = jnp.abs(d)
    return jnp.where(a < 1.0, 0.5 * d * d, a - 0.5)


def _match_kernel(locT_ref, landmT_ref, tgt_ref, tgtT_ref,
                  priorsT_ref, pos_ref, np_ref, ll_ref, llm_ref,
                  *, num_priors: int, num_obj: int, pack: int):
    P = num_priors
    O = num_obj
    R = pack * O

    pr = priorsT_ref[0]
    pcx, pcy, pw, ph = pr[0:1, :], pr[1:2, :], pr[2:3, :], pr[3:4, :]
    px1 = pcx - pw * 0.5
    py1 = pcy - ph * 0.5
    px2 = pcx + pw * 0.5
    py2 = pcy + ph * 0.5
    area_b = (px2 - px1) * (py2 - py1)

    tg = tgt_ref[:].reshape(R, 15)
    tx1 = tg[:, 0:1]
    ty1 = tg[:, 1:2]
    tx2 = tg[:, 2:3]
    ty2 = tg[:, 3:4]
    area_a = (tx2 - tx1) * (ty2 - ty1)

    ix1 = jnp.maximum(tx1, px1)
    iy1 = jnp.maximum(ty1, py1)
    ix2 = jnp.minimum(tx2, px2)
    iy2 = jnp.minimum(ty2, py2)
    inter = jnp.maximum(ix2 - ix1, 0.0) * jnp.maximum(iy2 - iy1, 0.0)
    ov = inter / (area_a + area_b - inter)

    idx_p = jax.lax.broadcasted_iota(jnp.int32, (R, P), 1)
    idx_t = jax.lax.broadcasted_iota(jnp.int32, (O, P), 0)

    bpo = jnp.max(ov, axis=1, keepdims=True)
    bp_idx = jnp.min(jnp.where(ov == bpo, idx_p, P), axis=1, keepdims=True)
    valid = bpo >= 0.2
    is_bp = bp_idx == idx_p

    row10 = jax.lax.broadcasted_iota(jnp.int32, (10, P), 0)
    is_x = (row10 % 2) == 0
    cxy10 = jnp.where(is_x, pcx, pcy)
    wh10 = jnp.where(is_x, pw, ph)
    pcxy = pr[0:2, :]
    pwh = pr[2:4, :]

    for i in range(pack):
        ovi = ov[O * i:O * (i + 1), :]
        validi = valid[O * i:O * (i + 1), :]
        is_bpi = is_bp[O * i:O * (i + 1), :]
        any_valid = jnp.max(validi.astype(jnp.float32)) > 0.0

        bto = jnp.max(ovi, axis=0, keepdims=True)
        bt_idx = jnp.min(jnp.where(ovi == bto, idx_t, O), axis=0,
                         keepdims=True)

        has_valid_ov = jnp.max(jnp.where(is_bpi & validi, 1.0, 0.0), axis=0,
                               keepdims=True) > 0.0
        new_ov = jnp.where(has_valid_ov, 2.0, bto)
        t_over = jnp.max(jnp.where(is_bpi, idx_t, -1), axis=0, keepdims=True)
        new_idx = jnp.where(t_over >= 0, t_over, bt_idx)

        onehot = (new_idx == idx_t).astype(jnp.float32)
        matched = jnp.dot(tgtT_ref[i], onehot,
                          preferred_element_type=jnp.float32)

        label = matched[14:15, :]
        conf = jnp.where(new_ov < _THRESHOLD, 0.0, label)
        conf = jnp.where(any_valid, conf, 0.0)
        pos = conf > 0.5
        posf = pos.astype(jnp.float32)

        m01 = matched[0:2, :]
        m23 = matched[2:4, :]
        g_cxy = ((m01 + m23) * 0.5 - pcxy) / (_VAR0 * pwh)
        g_wh = jnp.log((m23 - m01) / pwh) / _VAR1
        locT = locT_ref[i]
        loss_l = jnp.sum(posf * (_smooth_l1(locT[0:2, :] - g_cxy)
                                 + _smooth_l1(locT[2:4, :] - g_wh)))

        g_lm = (matched[4:14, :] - cxy10) / (_VAR0 * wh10)
        loss_lm = jnp.sum(posf * _smooth_l1(landmT_ref[i] - g_lm))

        pos_ref[i] = posf
        np_ref[i] = jnp.full((1, 1), jnp.sum(posf), jnp.float32)
        ll_ref[i] = jnp.full((1, 1), loss_l, jnp.float32)
        llm_ref[i] = jnp.full((1, 1), loss_lm, jnp.float32)


def _mine_kernel(pos_ref, c0_ref, c1_ref, np_ref, ll_ref, llm_ref,
                 out_l_ref, out_c_ref, out_lm_ref,
                 *, batch: int, num_priors: int):
    P = num_priors
    posf = pos_ref[:]
    pos = posf > 0.5
    c0 = c0_ref[:]
    c1 = c1_ref[:]
    m = jnp.maximum(c0, c1)
    lse_tail = jnp.log(1.0 + jnp.exp(-jnp.abs(c0 - c1)))
    picked = jnp.where(pos, c1, c0)
    ce = jnp.maximum((m - picked) + lse_tail, 0.0)
    cep = jnp.sum(posf * ce)
    x = jnp.where(pos, 0.0, ce)
    xi = jax.lax.bitcast_convert_type(x, jnp.int32)
    npos = np_ref[:, :, 0]
    k = jnp.minimum(npos.astype(jnp.int32) * _NEGPOS_RATIO, P - 1)

    def body(i, t):
        cand = t | (jnp.int32(1) << (jnp.int32(30) - i))
        cnt = jnp.sum((xi >= cand).astype(jnp.int32), axis=1, keepdims=True)
        return jnp.where(cnt >= k, cand, t)

    t = jax.lax.fori_loop(0, 21, body, jnp.zeros((batch, 1), jnp.int32))
    v = jax.lax.bitcast_convert_type(t, jnp.float32)
    gt = x > v
    sum_gt = jnp.sum(jnp.where(gt, x, 0.0), axis=1, keepdims=True)
    cnt_gt = jnp.sum(gt.astype(jnp.int32), axis=1, keepdims=True)
    row = sum_gt + (k - cnt_gt).astype(jnp.float32) * v
    mining = jnp.sum(jnp.where(k > 0, row, 0.0))

    n = jnp.maximum(jnp.sum(npos), 1.0)
    out_l_ref[:, :] = jnp.full((1, 1), jnp.sum(ll_ref[:]) / n, jnp.float32)
    out_c_ref[:, :] = jnp.full((1, 1), (cep + mining) / n, jnp.float32)
    out_lm_ref[:, :] = jnp.full((1, 1), jnp.sum(llm_ref[:]) / n, jnp.float32)


@jax.jit
def kernel(locations_data, confidence_data, landmark_data, targets, priors):
    B, P, _ = locations_data.shape
    O = targets.shape[1]

    locT = locations_data.transpose(0, 2, 1)
    landmT = landmark_data.transpose(0, 2, 1)
    tgtT = targets.transpose(0, 2, 1)
    priorsT = priors.T[None]
    c0 = confidence_data[:, :, 0]
    c1 = confidence_data[:, :, 1]

    PACK = 4
    s111 = jax.ShapeDtypeStruct((B, 1, 1), jnp.float32)
    k1 = functools.partial(_match_kernel, num_priors=P, num_obj=O, pack=PACK)
    posf, npos, ll, llm = pl.pallas_call(
        k1,
        grid=(B // PACK,),
        in_specs=[
            pl.BlockSpec((PACK, 4, P), lambda b: (b, 0, 0)),
            pl.BlockSpec((PACK, 10, P), lambda b: (b, 0, 0)),
            pl.BlockSpec((PACK, O, 15), lambda b: (b, 0, 0)),
            pl.BlockSpec((PACK, 15, O), lambda b: (b, 0, 0)),
            pl.BlockSpec((1, 4, P), lambda b: (0, 0, 0)),
        ],
        out_specs=[
            pl.BlockSpec((PACK, 1, P), lambda b: (b, 0, 0)),
            pl.BlockSpec((PACK, 1, 1), lambda b: (b, 0, 0)),
            pl.BlockSpec((PACK, 1, 1), lambda b: (b, 0, 0)),
            pl.BlockSpec((PACK, 1, 1), lambda b: (b, 0, 0)),
        ],
        out_shape=[
            jax.ShapeDtypeStruct((B, 1, P), jnp.float32),
            s111, s111, s111,
        ],
        compiler_params=pltpu.CompilerParams(
            dimension_semantics=("parallel",)),
    )(locT, landmT, targets, tgtT, priorsT)

    scalar = jax.ShapeDtypeStruct((1, 1), jnp.float32)
    k2 = functools.partial(_mine_kernel, batch=B, num_priors=P)
    out_l, out_c, out_lm = pl.pallas_call(
        k2,
        out_shape=[scalar, scalar, scalar],
    )(posf.reshape(B, P), c0, c1, npos, ll, llm)
    return out_l[0, 0], out_c[0, 0], out_lm[0, 0]
